# baseline (device time: 39685 ns/iter reference)
import jax
import jax.numpy as jnp
from jax import lax
from jax.experimental import pallas as pl
from jax.experimental.pallas import tpu as pltpu

M = 1024
HALF = M // 2
NC = 8
CW = M // NC
LAG = 2


def kernel(dy, W):
    k = dy.shape[1]

    def body(dy_ref, w_ref, out_ref,
             dyf32, dybf, wstage,
             xsend, xrecv, ysend, yrecv,
             dy_sem, w_sems,
             xsend_sems, xrecv_sems, ysend_sems, yrecv_sems):
        my_x = lax.axis_index("x")
        my_y = lax.axis_index("y")
        xnbr = (1 - my_x, my_y)
        ynbr = (my_x, 1 - my_y)
        row0 = my_y * HALF

        dy_dma = pltpu.make_async_copy(
            dy_ref.at[pl.ds(row0, HALF), :], dyf32, dy_sem)
        dy_dma.start()

        def w_dma(c):
            return pltpu.make_async_copy(
                w_ref.at[pl.ds(c * CW, CW), :], wstage.at[c % 2],
                w_sems.at[c % 2])

        w_dma(0).start()
        w_dma(1).start()

        barrier_sem = pltpu.get_barrier_semaphore()
        for nbr in (xnbr, ynbr):
            pl.semaphore_signal(
                barrier_sem, inc=1,
                device_id=nbr, device_id_type=pl.DeviceIdType.MESH,
            )
        pl.semaphore_wait(barrier_sem, 2)

        dy_dma.wait()
        dybf[:, :] = dyf32[:, :].astype(jnp.bfloat16)

        def rdma_x(c):
            return pltpu.make_async_remote_copy(
                src_ref=xsend.at[c], dst_ref=xrecv.at[c],
                send_sem=xsend_sems.at[c], recv_sem=xrecv_sems.at[c],
                device_id=xnbr, device_id_type=pl.DeviceIdType.MESH,
            )

        def rdma_y(c):
            return pltpu.make_async_remote_copy(
                src_ref=ysend.at[c], dst_ref=yrecv.at[c],
                send_sem=ysend_sems.at[c], recv_sem=yrecv_sems.at[c],
                device_id=ynbr, device_id_type=pl.DeviceIdType.MESH,
            )

        def reduce_and_forward(d):
            rdma_x(d).wait_recv()
            red = (xsend[d, :, :].astype(jnp.float32)
                   + xrecv[d, :, :].astype(jnp.float32))
            out_ref[pl.ds(row0, HALF), pl.ds(d * CW, CW)] = red
            ysend[d, :, :] = red.astype(jnp.bfloat16)
            rdma_y(d).start()

        for c in range(NC):
            w_dma(c).wait()
            p = lax.dot_general(
                dybf[:, :], wstage[c % 2].astype(jnp.bfloat16),
                dimension_numbers=(((1,), (1,)), ((), ())),
                preferred_element_type=jnp.float32,
            )
            if c + 2 < NC:
                w_dma(c + 2).start()
            xsend[c, :, :] = p.astype(jnp.bfloat16)
            rdma_x(c).start()
            if c >= LAG:
                reduce_and_forward(c - LAG)
        for d in range(NC - LAG, NC):
            reduce_and_forward(d)

        other0 = (1 - my_y) * HALF
        for c in range(NC):
            rdma_y(c).wait_recv()
            out_ref[pl.ds(other0, HALF), pl.ds(c * CW, CW)] = (
                yrecv[c, :, :].astype(jnp.float32)
            )

        for c in range(NC):
            rdma_x(c).wait_send()
            rdma_y(c).wait_send()

    return pl.pallas_call(
        body,
        out_shape=jax.ShapeDtypeStruct((M, M), jnp.float32),
        in_specs=[
            pl.BlockSpec(memory_space=pl.ANY),
            pl.BlockSpec(memory_space=pl.ANY),
        ],
        out_specs=pl.BlockSpec(memory_space=pltpu.VMEM),
        scratch_shapes=[
            pltpu.VMEM((HALF, k), jnp.float32),
            pltpu.VMEM((HALF, k), jnp.bfloat16),
            pltpu.VMEM((2, CW, k), jnp.float32),
            pltpu.VMEM((NC, HALF, CW), jnp.bfloat16),
            pltpu.VMEM((NC, HALF, CW), jnp.bfloat16),
            pltpu.VMEM((NC, HALF, CW), jnp.bfloat16),
            pltpu.VMEM((NC, HALF, CW), jnp.bfloat16),
            pltpu.SemaphoreType.DMA,
            pltpu.SemaphoreType.DMA((2,)),
            pltpu.SemaphoreType.DMA((NC,)),
            pltpu.SemaphoreType.DMA((NC,)),
            pltpu.SemaphoreType.DMA((NC,)),
            pltpu.SemaphoreType.DMA((NC,)),
        ],
        compiler_params=pltpu.CompilerParams(
            collective_id=0,
            vmem_limit_bytes=100 * 1024 * 1024,
        ),
    )(dy, W)


# device time: 39096 ns/iter; 1.0151x vs baseline; 1.0151x over previous
import jax
import jax.numpy as jnp
from jax import lax
from jax.experimental import pallas as pl
from jax.experimental.pallas import tpu as pltpu

M = 1024
HALF = M // 2
NC = 8
CW = M // NC
LAG = 2


def kernel(dy, W):
    k = dy.shape[1]

    def body(dy_ref, w_ref, out_ref,
             dyf32, wstage,
             xsend, xrecv, ysend, yrecv,
             dy_sem, w_sems,
             xsend_sems, xrecv_sems, ysend_sems, yrecv_sems):
        my_x = lax.axis_index("x")
        my_y = lax.axis_index("y")
        xnbr = (1 - my_x, my_y)
        ynbr = (my_x, 1 - my_y)
        row0 = my_y * HALF

        def w_dma(c):
            return pltpu.make_async_copy(
                w_ref.at[pl.ds(c * CW, CW), :], wstage.at[c % 2],
                w_sems.at[c % 2])

        w_dma(0).start()
        dy_dma = pltpu.make_async_copy(
            dy_ref.at[pl.ds(row0, HALF), :], dyf32, dy_sem)
        dy_dma.start()
        w_dma(1).start()

        barrier_sem = pltpu.get_barrier_semaphore()
        for nbr in (xnbr, ynbr):
            pl.semaphore_signal(
                barrier_sem, inc=1,
                device_id=nbr, device_id_type=pl.DeviceIdType.MESH,
            )
        pl.semaphore_wait(barrier_sem, 2)

        dy_dma.wait()

        def rdma_x(c):
            return pltpu.make_async_remote_copy(
                src_ref=xsend.at[c], dst_ref=xrecv.at[c],
                send_sem=xsend_sems.at[c], recv_sem=xrecv_sems.at[c],
                device_id=xnbr, device_id_type=pl.DeviceIdType.MESH,
            )

        def rdma_y(c):
            return pltpu.make_async_remote_copy(
                src_ref=ysend.at[c], dst_ref=yrecv.at[c],
                send_sem=ysend_sems.at[c], recv_sem=yrecv_sems.at[c],
                device_id=ynbr, device_id_type=pl.DeviceIdType.MESH,
            )

        def reduce_and_forward(d):
            rdma_x(d).wait_recv()
            red = (xsend[d, :, :].astype(jnp.float32)
                   + xrecv[d, :, :].astype(jnp.float32))
            out_ref[pl.ds(row0, HALF), pl.ds(d * CW, CW)] = red
            ysend[d, :, :] = red.astype(jnp.bfloat16)
            rdma_y(d).start()

        for c in range(NC):
            w_dma(c).wait()
            p = lax.dot_general(
                dyf32[:, :], wstage[c % 2],
                dimension_numbers=(((1,), (1,)), ((), ())),
                preferred_element_type=jnp.float32,
            )
            if c + 2 < NC:
                w_dma(c + 2).start()
            xsend[c, :, :] = p.astype(jnp.bfloat16)
            rdma_x(c).start()
            if c >= LAG:
                reduce_and_forward(c - LAG)
        for d in range(NC - LAG, NC):
            reduce_and_forward(d)

        other0 = (1 - my_y) * HALF
        for c in range(NC):
            rdma_y(c).wait_recv()
            out_ref[pl.ds(other0, HALF), pl.ds(c * CW, CW)] = (
                yrecv[c, :, :].astype(jnp.float32)
            )

        for c in range(NC):
            rdma_x(c).wait_send()
            rdma_y(c).wait_send()

    return pl.pallas_call(
        body,
        out_shape=jax.ShapeDtypeStruct((M, M), jnp.float32),
        in_specs=[
            pl.BlockSpec(memory_space=pl.ANY),
            pl.BlockSpec(memory_space=pl.ANY),
        ],
        out_specs=pl.BlockSpec(memory_space=pltpu.VMEM),
        scratch_shapes=[
            pltpu.VMEM((HALF, k), jnp.float32),
            pltpu.VMEM((2, CW, k), jnp.float32),
            pltpu.VMEM((NC, HALF, CW), jnp.bfloat16),
            pltpu.VMEM((NC, HALF, CW), jnp.bfloat16),
            pltpu.VMEM((NC, HALF, CW), jnp.bfloat16),
            pltpu.VMEM((NC, HALF, CW), jnp.bfloat16),
            pltpu.SemaphoreType.DMA,
            pltpu.SemaphoreType.DMA((2,)),
            pltpu.SemaphoreType.DMA((NC,)),
            pltpu.SemaphoreType.DMA((NC,)),
            pltpu.SemaphoreType.DMA((NC,)),
            pltpu.SemaphoreType.DMA((NC,)),
        ],
        compiler_params=pltpu.CompilerParams(
            collective_id=0,
            vmem_limit_bytes=100 * 1024 * 1024,
        ),
    )(dy, W)


# device time: 34885 ns/iter; 1.1376x vs baseline; 1.1207x over previous
import jax
import jax.numpy as jnp
from jax import lax
from jax.experimental import pallas as pl
from jax.experimental.pallas import tpu as pltpu

M = 1024
HALF = M // 2
NC = 4
CW = M // NC
LAG = 1


def kernel(dy, W):
    k = dy.shape[1]

    def body(dy_ref, w_ref, out_ref,
             dyf32, wstage,
             xsend, xrecv, ysend, yrecv,
             dy_sem, w_sems,
             xsend_sems, xrecv_sems, ysend_sems, yrecv_sems):
        my_x = lax.axis_index("x")
        my_y = lax.axis_index("y")
        xnbr = (1 - my_x, my_y)
        ynbr = (my_x, 1 - my_y)
        row0 = my_y * HALF

        def w_dma(c):
            return pltpu.make_async_copy(
                w_ref.at[pl.ds(c * CW, CW), :], wstage.at[c % 2],
                w_sems.at[c % 2])

        w_dma(0).start()
        dy_dma = pltpu.make_async_copy(
            dy_ref.at[pl.ds(row0, HALF), :], dyf32, dy_sem)
        dy_dma.start()
        w_dma(1).start()

        barrier_sem = pltpu.get_barrier_semaphore()
        for nbr in (xnbr, ynbr):
            pl.semaphore_signal(
                barrier_sem, inc=1,
                device_id=nbr, device_id_type=pl.DeviceIdType.MESH,
            )
        pl.semaphore_wait(barrier_sem, 2)

        dy_dma.wait()

        def rdma_x(c):
            return pltpu.make_async_remote_copy(
                src_ref=xsend.at[c], dst_ref=xrecv.at[c],
                send_sem=xsend_sems.at[c], recv_sem=xrecv_sems.at[c],
                device_id=xnbr, device_id_type=pl.DeviceIdType.MESH,
            )

        def rdma_y(c):
            return pltpu.make_async_remote_copy(
                src_ref=ysend.at[c], dst_ref=yrecv.at[c],
                send_sem=ysend_sems.at[c], recv_sem=yrecv_sems.at[c],
                device_id=ynbr, device_id_type=pl.DeviceIdType.MESH,
            )

        def reduce_and_forward(d):
            rdma_x(d).wait_recv()
            red = (xsend[d, :, :].astype(jnp.float32)
                   + xrecv[d, :, :].astype(jnp.float32))
            out_ref[pl.ds(row0, HALF), pl.ds(d * CW, CW)] = red
            ysend[d, :, :] = red.astype(jnp.bfloat16)
            rdma_y(d).start()

        for c in range(NC):
            w_dma(c).wait()
            p = lax.dot_general(
                dyf32[:, :], wstage[c % 2],
                dimension_numbers=(((1,), (1,)), ((), ())),
                preferred_element_type=jnp.float32,
            )
            if c + 2 < NC:
                w_dma(c + 2).start()
            xsend[c, :, :] = p.astype(jnp.bfloat16)
            rdma_x(c).start()
            if c >= LAG:
                reduce_and_forward(c - LAG)
        for d in range(NC - LAG, NC):
            reduce_and_forward(d)

        other0 = (1 - my_y) * HALF
        for c in range(NC):
            rdma_y(c).wait_recv()
            out_ref[pl.ds(other0, HALF), pl.ds(c * CW, CW)] = (
                yrecv[c, :, :].astype(jnp.float32)
            )

        for c in range(NC):
            rdma_x(c).wait_send()
            rdma_y(c).wait_send()

    return pl.pallas_call(
        body,
        out_shape=jax.ShapeDtypeStruct((M, M), jnp.float32),
        in_specs=[
            pl.BlockSpec(memory_space=pl.ANY),
            pl.BlockSpec(memory_space=pl.ANY),
        ],
        out_specs=pl.BlockSpec(memory_space=pltpu.VMEM),
        scratch_shapes=[
            pltpu.VMEM((HALF, k), jnp.float32),
            pltpu.VMEM((2, CW, k), jnp.float32),
            pltpu.VMEM((NC, HALF, CW), jnp.bfloat16),
            pltpu.VMEM((NC, HALF, CW), jnp.bfloat16),
            pltpu.VMEM((NC, HALF, CW), jnp.bfloat16),
            pltpu.VMEM((NC, HALF, CW), jnp.bfloat16),
            pltpu.SemaphoreType.DMA,
            pltpu.SemaphoreType.DMA((2,)),
            pltpu.SemaphoreType.DMA((NC,)),
            pltpu.SemaphoreType.DMA((NC,)),
            pltpu.SemaphoreType.DMA((NC,)),
            pltpu.SemaphoreType.DMA((NC,)),
        ],
        compiler_params=pltpu.CompilerParams(
            collective_id=0,
            vmem_limit_bytes=100 * 1024 * 1024,
        ),
    )(dy, W)
